# baseline (speedup 1.0000x reference)
import jax
import jax.numpy as jnp
from jax.experimental import pallas as pl
from jax.experimental.pallas import tpu as pltpu

NQ = 8
K = 1024
D = 256
W = 2048


def _rvq_kernel(xtaps_ref, enc_w_ref, enc_b_ref, dec_w_ref,
                cbt_ref, cb_ref, y_ref, loss_ref):
    x = jax.lax.dot(xtaps_ref[...].astype(jnp.bfloat16),
                    enc_w_ref[...].astype(jnp.bfloat16),
                    preferred_element_type=jnp.float32) + enc_b_ref[...]
    residual = x
    loss = jnp.zeros((1, 1), jnp.float32)
    iota = jax.lax.broadcasted_iota(jnp.int32, (W, K), 1)
    for q in range(NQ):
        cbt = cbt_ref[q]
        cb = cb_ref[q]
        scores_m2 = jax.lax.dot(residual.astype(jnp.bfloat16),
                                cbt.astype(jnp.bfloat16),
                                preferred_element_type=jnp.float32)
        cnorm = 0.25 * jnp.sum(cbt * cbt, axis=0, keepdims=True)
        d = scores_m2 + cnorm
        m = jnp.min(d, axis=1, keepdims=True)
        idx = jnp.min(jnp.where(d == m, iota, K), axis=1, keepdims=True)
        onehot = (iota == idx).astype(jnp.bfloat16)
        cb_hi = cb.astype(jnp.bfloat16)
        r1 = cb - cb_hi.astype(jnp.float32)
        cb_mid = r1.astype(jnp.bfloat16)
        cb_lo = (r1 - cb_mid.astype(jnp.float32)).astype(jnp.bfloat16)
        qv = (jax.lax.dot(onehot, cb_hi, preferred_element_type=jnp.float32)
              + jax.lax.dot(onehot, cb_mid,
                            preferred_element_type=jnp.float32)
              + jax.lax.dot(onehot, cb_lo,
                            preferred_element_type=jnp.float32))
        residual = residual - qv
        loss = loss + jnp.sum(residual * residual, keepdims=True)
    quantized = x - residual
    y_ref[...] = jax.lax.dot(quantized, dec_w_ref[...],
                             precision=jax.lax.Precision.HIGHEST,
                             preferred_element_type=jnp.float32)
    loss_ref[0] = loss


@jax.jit
def kernel(x, enc_w, enc_b, dec_w, dec_b, codebooks):
    B, _, T = x.shape
    N = B * T
    nb = N // W
    xt = x[:, 0, :]
    left = jnp.pad(xt[:, :-1], ((0, 0), (1, 0)))
    right = jnp.pad(xt[:, 1:], ((0, 0), (0, 1)))
    xtaps = jnp.stack([left, xt, right], axis=-1).reshape(N, 3)
    enc_wr = enc_w[:, 0, :].T
    enc_b2 = enc_b[None, :]
    dec_wr = dec_w[0]
    cbt = -2.0 * jnp.transpose(codebooks, (0, 2, 1))

    y, loss_parts = pl.pallas_call(
        _rvq_kernel,
        grid=(nb,),
        in_specs=[
            pl.BlockSpec((W, 3), lambda i: (i, 0)),
            pl.BlockSpec((3, D), lambda i: (0, 0)),
            pl.BlockSpec((1, D), lambda i: (0, 0)),
            pl.BlockSpec((D, 3), lambda i: (0, 0)),
            pl.BlockSpec((NQ, D, K), lambda i: (0, 0, 0)),
            pl.BlockSpec((NQ, K, D), lambda i: (0, 0, 0)),
        ],
        out_specs=[
            pl.BlockSpec((W, 3), lambda i: (i, 0)),
            pl.BlockSpec((1, 1, 1), lambda i: (i, 0, 0)),
        ],
        out_shape=[
            jax.ShapeDtypeStruct((N, 3), jnp.float32),
            jax.ShapeDtypeStruct((nb, 1, 1), jnp.float32),
        ],
        compiler_params=pltpu.CompilerParams(
            dimension_semantics=("parallel",)),
    )(xtaps, enc_wr, enc_b2, dec_wr, cbt, codebooks)

    yb = y.reshape(B, T, 3)
    decoded = (yb[:, :, 1]
               + jnp.pad(yb[:, :-1, 0], ((0, 0), (1, 0)))
               + jnp.pad(yb[:, 1:, 2], ((0, 0), (0, 1))))
    decoded = decoded[:, None, :] + dec_b[None, :, None]
    commit_loss = jnp.sum(loss_parts) / jnp.float32(NQ * N * D)
    return decoded, commit_loss


# speedup vs baseline: 1.4157x; 1.4157x over previous
import jax
import jax.numpy as jnp
from jax.experimental import pallas as pl
from jax.experimental.pallas import tpu as pltpu

NQ = 8
K = 1024
D = 256
W = 2048


def _rvq_kernel(xtaps_ref, enc_w_ref, enc_b_ref, dec_w_ref,
                cbt_ref, cb_ref, y_ref, loss_ref):
    x = jax.lax.dot(xtaps_ref[...].astype(jnp.bfloat16),
                    enc_w_ref[...].astype(jnp.bfloat16),
                    preferred_element_type=jnp.float32) + enc_b_ref[...]
    loss = jnp.zeros((1, 1), jnp.float32)
    H = W // 2
    iota = jax.lax.broadcasted_iota(jnp.int32, (H, K), 1)
    halves = [x[:H], x[H:]]
    for q in range(NQ):
        cbt = cbt_ref[q]
        cb = cb_ref[q]
        cnorm = 0.25 * jnp.sum(cbt * cbt, axis=0, keepdims=True)
        cb_hi = cb.astype(jnp.bfloat16)
        r1 = cb - cb_hi.astype(jnp.float32)
        cb_mid = r1.astype(jnp.bfloat16)
        cb_lo = (r1 - cb_mid.astype(jnp.float32)).astype(jnp.bfloat16)
        new_halves = []
        for residual in halves:
            scores_m2 = jax.lax.dot(residual.astype(jnp.bfloat16),
                                    cbt.astype(jnp.bfloat16),
                                    preferred_element_type=jnp.float32)
            d = scores_m2 + cnorm
            m = jnp.min(d, axis=1, keepdims=True)
            idx = jnp.min(jnp.where(d == m, iota, K), axis=1, keepdims=True)
            onehot = (iota == idx).astype(jnp.bfloat16)
            qv = (jax.lax.dot(onehot, cb_hi,
                              preferred_element_type=jnp.float32)
                  + jax.lax.dot(onehot, cb_mid,
                                preferred_element_type=jnp.float32)
                  + jax.lax.dot(onehot, cb_lo,
                                preferred_element_type=jnp.float32))
            new_halves.append(residual - qv)
        halves = new_halves
        loss = (loss + jnp.sum(halves[0] * halves[0], keepdims=True)
                + jnp.sum(halves[1] * halves[1], keepdims=True))
    residual = jnp.concatenate(halves, axis=0)
    quantized = x - residual
    y_ref[...] = jax.lax.dot(quantized, dec_w_ref[...],
                             precision=jax.lax.Precision.HIGHEST,
                             preferred_element_type=jnp.float32)
    loss_ref[0] = loss


@jax.jit
def kernel(x, enc_w, enc_b, dec_w, dec_b, codebooks):
    B, _, T = x.shape
    N = B * T
    nb = N // W
    xt = x[:, 0, :]
    left = jnp.pad(xt[:, :-1], ((0, 0), (1, 0)))
    right = jnp.pad(xt[:, 1:], ((0, 0), (0, 1)))
    xtaps = jnp.stack([left, xt, right], axis=-1).reshape(N, 3)
    enc_wr = enc_w[:, 0, :].T
    enc_b2 = enc_b[None, :]
    dec_wr = dec_w[0]
    cbt = -2.0 * jnp.transpose(codebooks, (0, 2, 1))

    y, loss_parts = pl.pallas_call(
        _rvq_kernel,
        grid=(nb,),
        in_specs=[
            pl.BlockSpec((W, 3), lambda i: (i, 0)),
            pl.BlockSpec((3, D), lambda i: (0, 0)),
            pl.BlockSpec((1, D), lambda i: (0, 0)),
            pl.BlockSpec((D, 3), lambda i: (0, 0)),
            pl.BlockSpec((NQ, D, K), lambda i: (0, 0, 0)),
            pl.BlockSpec((NQ, K, D), lambda i: (0, 0, 0)),
        ],
        out_specs=[
            pl.BlockSpec((W, 3), lambda i: (i, 0)),
            pl.BlockSpec((1, 1, 1), lambda i: (i, 0, 0)),
        ],
        out_shape=[
            jax.ShapeDtypeStruct((N, 3), jnp.float32),
            jax.ShapeDtypeStruct((nb, 1, 1), jnp.float32),
        ],
        compiler_params=pltpu.CompilerParams(
            dimension_semantics=("parallel",)),
    )(xtaps, enc_wr, enc_b2, dec_wr, cbt, codebooks)

    yb = y.reshape(B, T, 3)
    decoded = (yb[:, :, 1]
               + jnp.pad(yb[:, :-1, 0], ((0, 0), (1, 0)))
               + jnp.pad(yb[:, 1:, 2], ((0, 0), (0, 1))))
    decoded = decoded[:, None, :] + dec_b[None, :, None]
    commit_loss = jnp.sum(loss_parts) / jnp.float32(NQ * N * D)
    return decoded, commit_loss
